# baseline (speedup 1.0000x reference)
import functools
import math

import jax
import jax.numpy as jnp
from jax import lax
from jax.experimental import pallas as pl
from jax.experimental.pallas import tpu as pltpu
from jax.experimental.pallas import tpu_sc as plsc

_NC = 2
_NS = 16
_NW = _NC * _NS
_L = 16
_EB = 80


def _node_stage(nf, na, w_sc2, w_lin1, block_n):
  n, d = nf.shape
  inv = 1.0 / math.sqrt(d)

  def body(nf_ref, na_ref, wsc_ref, wl1_ref, h_ref, sc_ref):
    nf_b = nf_ref[...]
    h_ref[...] = jnp.dot(nf_b, wl1_ref[...],
                         preferred_element_type=jnp.float32) * inv
    sc_ref[...] = na_ref[...] * (
        jnp.dot(nf_b, wsc_ref[...], preferred_element_type=jnp.float32) * inv)

  grid = (n // block_n,)
  return pl.pallas_call(
      body,
      grid=grid,
      in_specs=[
          pl.BlockSpec((block_n, d), lambda i: (i, 0)),
          pl.BlockSpec((block_n, 1), lambda i: (i, 0)),
          pl.BlockSpec((d, d), lambda i: (0, 0)),
          pl.BlockSpec((d, d), lambda i: (0, 0)),
      ],
      out_specs=[
          pl.BlockSpec((block_n, d), lambda i: (i, 0)),
          pl.BlockSpec((block_n, d), lambda i: (i, 0)),
      ],
      out_shape=[
          jax.ShapeDtypeStruct((n, d), jnp.float32),
          jax.ShapeDtypeStruct((n, d), jnp.float32),
      ],
  )(nf, na, w_sc2, w_lin1)


def _edge_weight_stage(ee, sh, w0, w1, w2, block_e, e_out):
  e, nb = ee.shape
  h_dim = w0.shape[1]
  d = w2.shape[1]
  inv_nb = 1.0 / math.sqrt(nb)
  inv_h = 1.0 / math.sqrt(h_dim)

  def body(ee_ref, sh_ref, w0_ref, w1_ref, w2_ref, out_ref):
    x = jax.nn.swish(jnp.dot(ee_ref[...], w0_ref[...],
                             preferred_element_type=jnp.float32) * inv_nb)
    x = jax.nn.swish(jnp.dot(x, w1_ref[...],
                             preferred_element_type=jnp.float32) * inv_h)
    out_ref[...] = (jnp.dot(x, w2_ref[...],
                            preferred_element_type=jnp.float32) * inv_h
                    ) * sh_ref[...]

  grid = (e_out // block_e,)
  return pl.pallas_call(
      body,
      grid=grid,
      in_specs=[
          pl.BlockSpec((block_e, nb), lambda i: (i, 0)),
          pl.BlockSpec((block_e, 1), lambda i: (i, 0)),
          pl.BlockSpec((nb, h_dim), lambda i: (0, 0)),
          pl.BlockSpec((h_dim, h_dim), lambda i: (0, 0)),
          pl.BlockSpec((h_dim, d), lambda i: (0, 0)),
      ],
      out_specs=pl.BlockSpec((block_e, d), lambda i: (i, 0)),
      out_shape=jax.ShapeDtypeStruct((e_out, d), jnp.float32),
  )(ee, sh, w0, w1, w2)


def _make_sc_stage(n_pad, d, nblk):
  rows_per_tile = n_pad // _NS
  chunks = []
  off = 0
  while off < rows_per_tile:
    cnt = min(_EB, rows_per_tile - off)
    chunks.append((off, cnt))
    off += cnt
  nbt = nblk // _NW
  mesh = plsc.VectorSubcoreMesh(core_axis_name="c", subcore_axis_name="s")

  @functools.partial(
      pl.kernel,
      out_type=jax.ShapeDtypeStruct((_NC, n_pad, d), jnp.float32),
      mesh=mesh,
      scratch_types=[
          pltpu.VMEM((_EB,), jnp.int32),
          pltpu.VMEM((_EB,), jnp.int32),
          pltpu.VMEM((_EB,), jnp.int32),
          pltpu.VMEM((_EB, d), jnp.float32),
          pltpu.VMEM((_EB, d), jnp.float32),
          pltpu.VMEM((_EB,), jnp.int32),
          pltpu.VMEM((_EB,), jnp.int32),
          pltpu.VMEM((_EB,), jnp.int32),
          pltpu.VMEM((_EB, d), jnp.float32),
          pltpu.VMEM((_EB, d), jnp.float32),
          pltpu.VMEM_SHARED((n_pad, d), jnp.float32),
          pltpu.SemaphoreType.DMA,
          pltpu.SemaphoreType.DMA,
          pltpu.SemaphoreType.DMA,
          pltpu.SemaphoreType.DMA,
          pltpu.SemaphoreType.DMA,
          pltpu.SemaphoreType.DMA,
      ],
  )
  def sc_k(h_hbm, w_hbm, src_hbm, dst_hbm, out_hbm,
           src_v, dst_v, dst2_v, rows_v, w_v, src_b, dst_b, dst2_b, rows_b, w_b,
           agg_sh, semi0, semg0, sems0, semi1, semg1, sems1):
    c = lax.axis_index("c")
    s = lax.axis_index("s")
    wid = s * _NC + c
    base = s * rows_per_tile

    zero = jnp.zeros((_L,), jnp.float32)

    def zrow(i, carry):
      for j in range(d // _L):
        rows_v[i, pl.ds(j * _L, _L)] = zero
      return carry

    lax.fori_loop(0, _EB, zrow, 0)
    for coff, cnt in chunks:
      pltpu.sync_copy(rows_v.at[pl.ds(0, cnt)],
                      agg_sh.at[pl.ds(base + coff, cnt)])
    plsc.subcore_barrier()

    tile_b0 = wid * nbt

    bufs = ((src_v, dst_v, dst2_v, rows_v, w_v, semi0, semg0, sems0),
            (src_b, dst_b, dst2_b, rows_b, w_b, semi1, semg1, sems1))

    def issue_i(t, buf):
      sv, dv = buf[0], buf[1]
      sem = buf[5]
      b = tile_b0 + t
      pltpu.async_copy(src_hbm.at[pl.ds(b * _EB, _EB)], sv, sem)
      pltpu.async_copy(dst_hbm.at[pl.ds(b * _EB, _EB)], dv, sem)

    def wait_i(buf):
      sv, dv = buf[0], buf[1]
      sem = buf[5]
      pltpu.make_async_copy(src_hbm.at[pl.ds(0, _EB)], sv, sem).wait()
      pltpu.make_async_copy(dst_hbm.at[pl.ds(0, _EB)], dv, sem).wait()

    def issue_g(t, buf):
      sv, rv, wv, sem = buf[0], buf[3], buf[4], buf[6]
      pltpu.async_copy(h_hbm.at[sv], rv, sem)
      pltpu.async_copy(w_hbm.at[tile_b0 + t], wv, sem)

    def wait_g(buf):
      sv, rv, wv, sem = buf[0], buf[3], buf[4], buf[6]
      pltpu.make_async_copy(h_hbm.at[sv], rv, sem).wait()
      pltpu.make_async_copy(w_hbm.at[tile_b0], wv, sem).wait()

    def compute(buf):
      dv, d2, rv, wv, sem = buf[1], buf[2], buf[3], buf[4], buf[7]

      @plsc.parallel_loop(0, _EB, unroll=2)
      def mrow(i):
        for j2 in range(d // _L):
          sl = pl.ds(j2 * _L, _L)
          rv[i, sl] = rv[i, sl] * wv[i, sl]

      for j2 in range(_EB // _L):
        sl = pl.ds(j2 * _L, _L)
        d2[sl] = dv[sl]
      pltpu.async_copy(rv, agg_sh.at[d2], sem, add=True)

    def wait_s(buf):
      d2, rv, sem = buf[2], buf[3], buf[7]
      pltpu.make_async_copy(rv, agg_sh.at[d2], sem).wait()

    issue_i(0, bufs[0])
    wait_i(bufs[0])
    issue_g(0, bufs[0])
    issue_i(1, bufs[1])

    def half(t, cur, nxt, n_g, n_i, drain):
      if n_g:
        wait_i(nxt)
        if drain:
          wait_s(nxt)
        issue_g(t + 1, nxt)
      wait_g(cur)
      compute(cur)
      if n_i:
        issue_i(t + 2, cur)

    half(0, bufs[0], bufs[1], True, True, False)
    half(1, bufs[1], bufs[0], True, True, True)

    def pair(k, carry):
      t0 = 2 * k
      half(t0, bufs[0], bufs[1], True, True, True)
      half(t0 + 1, bufs[1], bufs[0], True, True, True)
      return carry

    lax.fori_loop(1, nbt // 2 - 1, pair, 0)
    half(nbt - 2, bufs[0], bufs[1], True, False, True)
    half(nbt - 1, bufs[1], bufs[0], False, False, False)
    wait_s(bufs[0])
    wait_s(bufs[1])
    plsc.subcore_barrier()

    for coff, cnt in chunks:
      pltpu.sync_copy(agg_sh.at[pl.ds(base + coff, cnt)],
                      rows_v.at[pl.ds(0, cnt)])
      pltpu.sync_copy(rows_v.at[pl.ds(0, cnt)],
                      out_hbm.at[c, pl.ds(base + coff, cnt)])

  return sc_k


def _final_stage(aggs, sc, w_lin2, n_neighbors, block_n, n_out):
  _, _, d = aggs.shape
  n = n_out
  scale = 1.0 / (n_neighbors * math.sqrt(d))

  def body(agg_ref, sc_ref, wl2_ref, out_ref):
    a = agg_ref[0] + agg_ref[1]
    h2 = jnp.dot(a, wl2_ref[...],
                 preferred_element_type=jnp.float32) * scale + sc_ref[...]
    out_ref[...] = jax.nn.swish(h2)

  grid = (n // block_n,)
  return pl.pallas_call(
      body,
      grid=grid,
      in_specs=[
          pl.BlockSpec((_NC, block_n, d), lambda i: (0, i, 0)),
          pl.BlockSpec((block_n, d), lambda i: (i, 0)),
          pl.BlockSpec((d, d), lambda i: (0, 0)),
      ],
      out_specs=pl.BlockSpec((block_n, d), lambda i: (i, 0)),
      out_shape=jax.ShapeDtypeStruct((n, d), jnp.float32),
  )(aggs, sc, w_lin2)


def kernel(node_features, node_attributes, edge_sh, edge_src, edge_dst,
           edge_embedded, W_sc, W_lin1, W_fc0, W_fc1, W_fc2, W_lin2):
  n, d = node_features.shape
  e = edge_src.shape[0]
  nb = edge_embedded.shape[1]

  h, sc = _node_stage(node_features, node_attributes, W_sc[:, 0, :], W_lin1,
                      block_n=1000)

  n_pad = ((n + _NS * 8 - 1) // (_NS * 8)) * (_NS * 8)

  grain = _EB * _NW * 2
  e_pad = ((e + grain - 1) // grain) * grain
  src = edge_src.astype(jnp.int32)
  dst = edge_dst.astype(jnp.int32)
  if e_pad != e:
    pad = e_pad - e
    src = jnp.concatenate([src, jnp.zeros((pad,), jnp.int32)])
    dst = jnp.concatenate([dst, jnp.full((pad,), n_pad - 1, jnp.int32)])

  w_edge = _edge_weight_stage(edge_embedded, edge_sh, W_fc0, W_fc1, W_fc2,
                              block_e=5120, e_out=e_pad)
  nblk = e_pad // _EB

  aggs = _make_sc_stage(n_pad, d, nblk)(
      h,
      w_edge.reshape(nblk, _EB, d),
      src,
      dst,
  )

  return _final_stage(aggs, sc, W_lin2, 32.0, block_n=1000, n_out=n)


# speedup vs baseline: 1.0018x; 1.0018x over previous
import functools
import math

import jax
import jax.numpy as jnp
from jax import lax
from jax.experimental import pallas as pl
from jax.experimental.pallas import tpu as pltpu
from jax.experimental.pallas import tpu_sc as plsc

_NC = 2
_NS = 16
_NW = _NC * _NS
_L = 16
_EB = 80


def _node_stage(nf, na, w_sc2, w_lin1, block_n):
  n, d = nf.shape
  inv = 1.0 / math.sqrt(d)

  def body(nf_ref, na_ref, wsc_ref, wl1_ref, h_ref, sc_ref):
    nf_b = nf_ref[...]
    h_ref[...] = jnp.dot(nf_b, wl1_ref[...],
                         preferred_element_type=jnp.float32) * inv
    sc_ref[...] = na_ref[...] * (
        jnp.dot(nf_b, wsc_ref[...], preferred_element_type=jnp.float32) * inv)

  grid = (n // block_n,)
  return pl.pallas_call(
      body,
      grid=grid,
      in_specs=[
          pl.BlockSpec((block_n, d), lambda i: (i, 0)),
          pl.BlockSpec((block_n, 1), lambda i: (i, 0)),
          pl.BlockSpec((d, d), lambda i: (0, 0)),
          pl.BlockSpec((d, d), lambda i: (0, 0)),
      ],
      out_specs=[
          pl.BlockSpec((block_n, d), lambda i: (i, 0)),
          pl.BlockSpec((block_n, d), lambda i: (i, 0)),
      ],
      out_shape=[
          jax.ShapeDtypeStruct((n, d), jnp.float32),
          jax.ShapeDtypeStruct((n, d), jnp.float32),
      ],
  )(nf, na, w_sc2, w_lin1)


def _edge_weight_stage(ee, sh, w0, w1, w2, block_e, e_out):
  e, nb = ee.shape
  h_dim = w0.shape[1]
  d = w2.shape[1]
  inv_nb = 1.0 / math.sqrt(nb)
  inv_h = 1.0 / math.sqrt(h_dim)

  def body(ee_ref, sh_ref, w0_ref, w1_ref, w2_ref, out_ref):
    x = jax.nn.swish(jnp.dot(ee_ref[...], w0_ref[...],
                             preferred_element_type=jnp.float32) * inv_nb)
    x = jax.nn.swish(jnp.dot(x, w1_ref[...],
                             preferred_element_type=jnp.float32) * inv_h)
    out_ref[...] = (jnp.dot(x, w2_ref[...],
                            preferred_element_type=jnp.float32) * inv_h
                    ) * sh_ref[...]

  grid = (e_out // block_e,)
  return pl.pallas_call(
      body,
      grid=grid,
      in_specs=[
          pl.BlockSpec((block_e, nb), lambda i: (i, 0)),
          pl.BlockSpec((block_e, 1), lambda i: (i, 0)),
          pl.BlockSpec((nb, h_dim), lambda i: (0, 0)),
          pl.BlockSpec((h_dim, h_dim), lambda i: (0, 0)),
          pl.BlockSpec((h_dim, d), lambda i: (0, 0)),
      ],
      out_specs=pl.BlockSpec((block_e, d), lambda i: (i, 0)),
      out_shape=jax.ShapeDtypeStruct((e_out, d), jnp.float32),
  )(ee, sh, w0, w1, w2)


def _make_sc_stage(n_pad, d, nblk):
  rows_per_tile = n_pad // _NS
  chunks = []
  off = 0
  while off < rows_per_tile:
    cnt = min(_EB, rows_per_tile - off)
    chunks.append((off, cnt))
    off += cnt
  nbt = nblk // _NW
  mesh = plsc.VectorSubcoreMesh(core_axis_name="c", subcore_axis_name="s")

  @functools.partial(
      pl.kernel,
      out_type=jax.ShapeDtypeStruct((_NC, n_pad, d), jnp.float32),
      mesh=mesh,
      scratch_types=[
          pltpu.VMEM((_EB,), jnp.int32),
          pltpu.VMEM((_EB,), jnp.int32),
          pltpu.VMEM((_EB,), jnp.int32),
          pltpu.VMEM((_EB, d), jnp.float32),
          pltpu.VMEM((_EB, d), jnp.float32),
          pltpu.VMEM((_EB,), jnp.int32),
          pltpu.VMEM((_EB,), jnp.int32),
          pltpu.VMEM((_EB,), jnp.int32),
          pltpu.VMEM((_EB, d), jnp.float32),
          pltpu.VMEM((_EB, d), jnp.float32),
          pltpu.VMEM_SHARED((n_pad, d), jnp.float32),
          pltpu.SemaphoreType.DMA,
          pltpu.SemaphoreType.DMA,
          pltpu.SemaphoreType.DMA,
          pltpu.SemaphoreType.DMA,
          pltpu.SemaphoreType.DMA,
          pltpu.SemaphoreType.DMA,
      ],
  )
  def sc_k(h_hbm, w_hbm, src_hbm, dst_hbm, out_hbm,
           src_v, dst_v, dst2_v, rows_v, w_v, src_b, dst_b, dst2_b, rows_b, w_b,
           agg_sh, semi0, semg0, sems0, semi1, semg1, sems1):
    c = lax.axis_index("c")
    s = lax.axis_index("s")
    wid = s * _NC + c
    base = s * rows_per_tile

    zero = jnp.zeros((_L,), jnp.float32)

    def zrow(i, carry):
      for j in range(d // _L):
        rows_v[i, pl.ds(j * _L, _L)] = zero
      return carry

    lax.fori_loop(0, _EB, zrow, 0)
    for coff, cnt in chunks:
      pltpu.sync_copy(rows_v.at[pl.ds(0, cnt)],
                      agg_sh.at[pl.ds(base + coff, cnt)])
    plsc.subcore_barrier()

    tile_b0 = wid * nbt

    bufs = ((src_v, dst_v, dst2_v, rows_v, w_v, semi0, semg0, sems0),
            (src_b, dst_b, dst2_b, rows_b, w_b, semi1, semg1, sems1))

    def issue_i(t, buf):
      sv, dv = buf[0], buf[1]
      sem = buf[5]
      b = tile_b0 + t
      pltpu.async_copy(src_hbm.at[pl.ds(b * _EB, _EB)], sv, sem)
      pltpu.async_copy(dst_hbm.at[pl.ds(b * _EB, _EB)], dv, sem)

    def wait_i(buf):
      sv, dv = buf[0], buf[1]
      sem = buf[5]
      pltpu.make_async_copy(src_hbm.at[pl.ds(0, _EB)], sv, sem).wait()
      pltpu.make_async_copy(dst_hbm.at[pl.ds(0, _EB)], dv, sem).wait()

    def issue_g(t, buf):
      sv, rv, wv, sem = buf[0], buf[3], buf[4], buf[6]
      pltpu.async_copy(h_hbm.at[sv], rv, sem)
      pltpu.async_copy(w_hbm.at[tile_b0 + t], wv, sem)

    def wait_g(buf):
      sv, rv, wv, sem = buf[0], buf[3], buf[4], buf[6]
      pltpu.make_async_copy(h_hbm.at[sv], rv, sem).wait()
      pltpu.make_async_copy(w_hbm.at[tile_b0], wv, sem).wait()

    def compute(buf):
      dv, d2, rv, wv, sem = buf[1], buf[2], buf[3], buf[4], buf[7]

      @plsc.parallel_loop(0, _EB, unroll=4)
      def mrow(i):
        for j2 in range(d // _L):
          sl = pl.ds(j2 * _L, _L)
          rv[i, sl] = rv[i, sl] * wv[i, sl]

      for j2 in range(_EB // _L):
        sl = pl.ds(j2 * _L, _L)
        d2[sl] = dv[sl]
      pltpu.async_copy(rv, agg_sh.at[d2], sem, add=True)

    def wait_s(buf):
      d2, rv, sem = buf[2], buf[3], buf[7]
      pltpu.make_async_copy(rv, agg_sh.at[d2], sem).wait()

    issue_i(0, bufs[0])
    wait_i(bufs[0])
    issue_g(0, bufs[0])
    issue_i(1, bufs[1])

    def half(t, cur, nxt, n_g, n_i, drain):
      if n_g:
        wait_i(nxt)
        if drain:
          wait_s(nxt)
        issue_g(t + 1, nxt)
      wait_g(cur)
      compute(cur)
      if n_i:
        issue_i(t + 2, cur)

    half(0, bufs[0], bufs[1], True, True, False)
    half(1, bufs[1], bufs[0], True, True, True)

    def pair(k, carry):
      t0 = 2 * k
      half(t0, bufs[0], bufs[1], True, True, True)
      half(t0 + 1, bufs[1], bufs[0], True, True, True)
      return carry

    lax.fori_loop(1, nbt // 2 - 1, pair, 0)
    half(nbt - 2, bufs[0], bufs[1], True, False, True)
    half(nbt - 1, bufs[1], bufs[0], False, False, False)
    wait_s(bufs[0])
    wait_s(bufs[1])
    plsc.subcore_barrier()

    for coff, cnt in chunks:
      pltpu.sync_copy(agg_sh.at[pl.ds(base + coff, cnt)],
                      rows_v.at[pl.ds(0, cnt)])
      pltpu.sync_copy(rows_v.at[pl.ds(0, cnt)],
                      out_hbm.at[c, pl.ds(base + coff, cnt)])

  return sc_k


def _final_stage(aggs, sc, w_lin2, n_neighbors, block_n, n_out):
  _, _, d = aggs.shape
  n = n_out
  scale = 1.0 / (n_neighbors * math.sqrt(d))

  def body(agg_ref, sc_ref, wl2_ref, out_ref):
    a = agg_ref[0] + agg_ref[1]
    h2 = jnp.dot(a, wl2_ref[...],
                 preferred_element_type=jnp.float32) * scale + sc_ref[...]
    out_ref[...] = jax.nn.swish(h2)

  grid = (n // block_n,)
  return pl.pallas_call(
      body,
      grid=grid,
      in_specs=[
          pl.BlockSpec((_NC, block_n, d), lambda i: (0, i, 0)),
          pl.BlockSpec((block_n, d), lambda i: (i, 0)),
          pl.BlockSpec((d, d), lambda i: (0, 0)),
      ],
      out_specs=pl.BlockSpec((block_n, d), lambda i: (i, 0)),
      out_shape=jax.ShapeDtypeStruct((n, d), jnp.float32),
  )(aggs, sc, w_lin2)


def kernel(node_features, node_attributes, edge_sh, edge_src, edge_dst,
           edge_embedded, W_sc, W_lin1, W_fc0, W_fc1, W_fc2, W_lin2):
  n, d = node_features.shape
  e = edge_src.shape[0]
  nb = edge_embedded.shape[1]

  h, sc = _node_stage(node_features, node_attributes, W_sc[:, 0, :], W_lin1,
                      block_n=1000)

  n_pad = ((n + _NS * 8 - 1) // (_NS * 8)) * (_NS * 8)

  grain = _EB * _NW * 2
  e_pad = ((e + grain - 1) // grain) * grain
  src = edge_src.astype(jnp.int32)
  dst = edge_dst.astype(jnp.int32)
  if e_pad != e:
    pad = e_pad - e
    src = jnp.concatenate([src, jnp.zeros((pad,), jnp.int32)])
    dst = jnp.concatenate([dst, jnp.full((pad,), n_pad - 1, jnp.int32)])

  w_edge = _edge_weight_stage(edge_embedded, edge_sh, W_fc0, W_fc1, W_fc2,
                              block_e=5120, e_out=e_pad)
  nblk = e_pad // _EB

  aggs = _make_sc_stage(n_pad, d, nblk)(
      h,
      w_edge.reshape(nblk, _EB, d),
      src,
      dst,
  )

  return _final_stage(aggs, sc, W_lin2, 32.0, block_n=1000, n_out=n)
